# baseline (speedup 1.0000x reference)
import functools

import jax
import jax.numpy as jnp
from jax import lax
from jax.experimental import pallas as pl
from jax.experimental.pallas import tpu as pltpu
from jax.experimental.pallas import tpu_sc as plsc

B, H, D = 4096, 26, 64
N_PER_HEAD = 100000
NC, NS, L = 2, 16, 16
NW = NC * NS
TOTAL = B * H
IDX_W = 128
ROWS_PER_W = TOTAL // NW
J_PER_W = ROWS_PER_W // IDX_W


def _sc_body(ids_hbm, table_hbm, out_hbm, idx_v, rows_v, sem0, sem1):
    wid = lax.axis_index("c") * NS + lax.axis_index("s")
    base = wid * ROWS_PER_W

    pltpu.sync_copy(ids_hbm.at[pl.ds(base, ROWS_PER_W)], idx_v)

    lanes = lax.iota(jnp.int32, L)

    def shift_vec(t, _):
        pos = base + t * L + lanes
        head = lax.rem(pos, H)
        sl = pl.ds(t * L, L)
        idx_v[sl] = idx_v[sl] + head * N_PER_HEAD
        return 0

    lax.fori_loop(0, ROWS_PER_W // L, shift_vec, 0)

    sems = (sem0, sem1)

    def start(j, b):
        pltpu.make_async_copy(table_hbm.at[idx_v.at[pl.ds(j * IDX_W, IDX_W)]],
                              rows_v.at[b], sems[b]).start()

    def drain(j, b):
        pltpu.make_async_copy(table_hbm.at[idx_v.at[pl.ds(j * IDX_W, IDX_W)]],
                              rows_v.at[b], sems[b]).wait()
        pltpu.sync_copy(rows_v.at[b], out_hbm.at[pl.ds(base + j * IDX_W, IDX_W)])

    start(0, 0)
    start(1, 1)

    def step(t, _):
        for b in range(2):
            drain(2 * t + b, b)
            start(2 * t + b + 2, b)
        return 0

    lax.fori_loop(0, J_PER_W // 2 - 1, step, 0, unroll=False)
    for b in range(2):
        drain(J_PER_W - 2 + b, b)


@functools.partial(
    pl.kernel,
    out_type=jax.ShapeDtypeStruct((TOTAL, D), jnp.float32),
    mesh=plsc.VectorSubcoreMesh(core_axis_name="c", subcore_axis_name="s"),
    compiler_params=pltpu.CompilerParams(use_tc_tiling_on_sc=False),
    scratch_types=[
        pltpu.VMEM((ROWS_PER_W,), jnp.int32),
        pltpu.VMEM((2, IDX_W, D), jnp.float32),
        pltpu.SemaphoreType.DMA,
        pltpu.SemaphoreType.DMA,
    ],
)
def _mhe_gather(ids_hbm, table_hbm, out_hbm, idx_v, rows_v, sem0, sem1):
    _sc_body(ids_hbm, table_hbm, out_hbm, idx_v, rows_v, sem0, sem1)


def kernel(input_ids, table):
    ids_flat = input_ids.reshape(TOTAL).astype(jnp.int32)
    out = _mhe_gather(ids_flat, table)
    return out.reshape(B, H, D)
